# baseline (speedup 1.0000x reference)
import functools

import jax
import jax.numpy as jnp
from jax import lax
from jax.experimental import pallas as pl
from jax.experimental.pallas import tpu as pltpu
from jax.experimental.pallas import tpu_sc as plsc

N_NODES = 10000
N_EDGES = 320000
D = 128
NW = 16
B = 80
E_PER_TILE = N_EDGES // NW
C = E_PER_TILE // B
ROWS_PER_TILE = 624
STAGE_ROWS = 48
REM_BASE = 16 * ROWS_PER_TILE
REM_ROWS = N_NODES - REM_BASE

_mesh = plsc.VectorSubcoreMesh(core_axis_name="c", subcore_axis_name="s",
                               num_cores=1)


@functools.partial(
    pl.kernel,
    mesh=_mesh,
    out_type=(
        jax.ShapeDtypeStruct((N_NODES, D), jnp.float32),
        jax.ShapeDtypeStruct((N_NODES,), jnp.float32),
    ),
    scratch_types=[
        pltpu.VMEM((2, B), jnp.int32),
        pltpu.VMEM((2, B, D), jnp.float32),
        pltpu.VMEM((STAGE_ROWS, D), jnp.float32),
        pltpu.VMEM((B,), jnp.float32),
        pltpu.VMEM((ROWS_PER_TILE,), jnp.float32),
        pltpu.VMEM_SHARED((N_NODES, D), jnp.float32),
        pltpu.VMEM_SHARED((N_NODES,), jnp.float32),
        pltpu.SemaphoreType.DMA,
        pltpu.SemaphoreType.DMA,
        pltpu.SemaphoreType.DMA,
        pltpu.SemaphoreType.DMA,
    ],
)
def _sc_aggregate(nbr_hbm, idx_hbm, z_feat_hbm, ones1d_hbm, z_cnt1d_hbm,
                  sum_out, cnt_out, idx_v, rows_v, stage_v, ones_v,
                  cnt_stage_v, acc_sh, cnt_sh, sem0, sem1, semc, sems):
    sid = lax.axis_index("s")
    wid = sid
    base_n = sid * ROWS_PER_TILE

    def zero_acc():
        for k in range(ROWS_PER_TILE // STAGE_ROWS):
            o = base_n + k * STAGE_ROWS
            pltpu.sync_copy(stage_v, acc_sh.at[pl.ds(o, STAGE_ROWS)])
        pltpu.sync_copy(cnt_stage_v, cnt_sh.at[pl.ds(base_n, ROWS_PER_TILE)])

        @pl.when(sid == 15)
        def _zero_rem():
            pltpu.sync_copy(stage_v.at[pl.ds(0, REM_ROWS)],
                            acc_sh.at[pl.ds(REM_BASE, REM_ROWS)])
            pltpu.sync_copy(cnt_stage_v.at[pl.ds(0, REM_ROWS)],
                            cnt_sh.at[pl.ds(REM_BASE, REM_ROWS)])

    def copy_acc_out(out_hbm):
        for k in range(ROWS_PER_TILE // STAGE_ROWS):
            o = base_n + k * STAGE_ROWS
            pltpu.sync_copy(acc_sh.at[pl.ds(o, STAGE_ROWS)], stage_v)
            pltpu.sync_copy(stage_v, out_hbm.at[pl.ds(o, STAGE_ROWS)])

        @pl.when(sid == 15)
        def _copy_rem():
            pltpu.sync_copy(acc_sh.at[pl.ds(REM_BASE, REM_ROWS)],
                            stage_v.at[pl.ds(0, REM_ROWS)])
            pltpu.sync_copy(stage_v.at[pl.ds(0, REM_ROWS)],
                            out_hbm.at[pl.ds(REM_BASE, REM_ROWS)])

    def idx_copy(e, slot, sem):
        return pltpu.make_async_copy(
            idx_hbm.at[pl.ds(wid * E_PER_TILE + e * B, B)],
            idx_v.at[slot], sem)

    def row_copy(e, slot, sem):
        return pltpu.make_async_copy(nbr_hbm.at[wid, e], rows_v.at[slot], sem)

    pltpu.sync_copy(z_feat_hbm, stage_v)
    pltpu.sync_copy(ones1d_hbm, ones_v)
    pltpu.sync_copy(z_cnt1d_hbm, cnt_stage_v)
    zero_acc()
    plsc.subcore_barrier()

    idx_copy(0, 0, sem0).start()
    row_copy(0, 0, sem0).start()

    def body_feat(j2, carry):
        e0 = 2 * j2
        idx_copy(e0 + 1, 1, sem1).start()
        row_copy(e0 + 1, 1, sem1).start()
        idx_copy(e0, 0, sem0).wait()
        cnt0 = pltpu.async_copy(ones_v, cnt_sh.at[idx_v.at[0]], semc,
                                add=True)
        row_copy(e0, 0, sem0).wait()
        s0 = pltpu.async_copy(rows_v.at[0], acc_sh.at[idx_v.at[0]], sems,
                              add=True)
        idx_copy(e0 + 1, 1, sem1).wait()
        cnt1 = pltpu.async_copy(ones_v, cnt_sh.at[idx_v.at[1]], semc,
                                add=True)
        row_copy(e0 + 1, 1, sem1).wait()
        s1 = pltpu.async_copy(rows_v.at[1], acc_sh.at[idx_v.at[1]], sems,
                              add=True)
        s0.wait()
        cnt0.wait()

        @pl.when(j2 + 1 < C // 2)
        def _pref():
            idx_copy(e0 + 2, 0, sem0).start()
            row_copy(e0 + 2, 0, sem0).start()

        s1.wait()
        cnt1.wait()
        return carry

    lax.fori_loop(0, C // 2, body_feat, 0)
    plsc.subcore_barrier()
    copy_acc_out(sum_out)
    pltpu.sync_copy(cnt_sh.at[pl.ds(base_n, ROWS_PER_TILE)], cnt_stage_v)
    pltpu.sync_copy(cnt_stage_v, cnt_out.at[pl.ds(base_n, ROWS_PER_TILE)])

    @pl.when(sid == 15)
    def _copy_cnt_rem():
        pltpu.sync_copy(cnt_sh.at[pl.ds(REM_BASE, REM_ROWS)],
                        cnt_stage_v.at[pl.ds(0, REM_ROWS)])
        pltpu.sync_copy(cnt_stage_v.at[pl.ds(0, REM_ROWS)],
                        cnt_out.at[pl.ds(REM_BASE, REM_ROWS)])


_TC_BLOCK = 1000


def _tc_body(self_ref, s_ref, c_ref, w_ref, o_ref):
    deg = c_ref[...]
    x = (self_ref[...] + s_ref[...]) / (deg + 1.0)
    o_ref[...] = jnp.dot(x, w_ref[...], preferred_element_type=jnp.float32)


def kernel(self_feat, nbr_feat, relation_src_indices, W):
    idx = relation_src_indices.astype(jnp.int32)
    nbr = nbr_feat.reshape(NW, C, B, D)
    z_feat = jnp.zeros((STAGE_ROWS, D), jnp.float32)
    ones1d = jnp.ones((B,), jnp.float32)
    z_cnt1d = jnp.zeros((ROWS_PER_TILE,), jnp.float32)

    sums, cnts = _sc_aggregate(nbr, idx, z_feat, ones1d, z_cnt1d)
    cnts = cnts.reshape(N_NODES, 1)

    out = pl.pallas_call(
        _tc_body,
        grid=(N_NODES // _TC_BLOCK,),
        in_specs=[
            pl.BlockSpec((_TC_BLOCK, D), lambda i: (i, 0)),
            pl.BlockSpec((_TC_BLOCK, D), lambda i: (i, 0)),
            pl.BlockSpec((_TC_BLOCK, 1), lambda i: (i, 0)),
            pl.BlockSpec((D, D), lambda i: (0, 0)),
        ],
        out_specs=pl.BlockSpec((_TC_BLOCK, D), lambda i: (i, 0)),
        out_shape=jax.ShapeDtypeStruct((N_NODES, D), jnp.float32),
    )(self_feat, sums, cnts, W)
    return out


# speedup vs baseline: 1.0551x; 1.0551x over previous
import functools

import jax
import jax.numpy as jnp
from jax import lax
from jax.experimental import pallas as pl
from jax.experimental.pallas import tpu as pltpu
from jax.experimental.pallas import tpu_sc as plsc

N_NODES = 10000
N_EDGES = 320000
D = 128
NW = 16
B = 80
E_PER_TILE = N_EDGES // NW
C = E_PER_TILE // B
ROWS_PER_TILE = 624
STAGE_ROWS = 48
REM_BASE = 16 * ROWS_PER_TILE
REM_ROWS = N_NODES - REM_BASE

_mesh = plsc.VectorSubcoreMesh(core_axis_name="c", subcore_axis_name="s",
                               num_cores=1)


@functools.partial(
    pl.kernel,
    mesh=_mesh,
    out_type=(
        jax.ShapeDtypeStruct((N_NODES, D), jnp.float32),
        jax.ShapeDtypeStruct((N_NODES,), jnp.float32),
    ),
    scratch_types=[
        pltpu.VMEM((2, B), jnp.int32),
        pltpu.VMEM((2, B, D), jnp.float32),
        pltpu.VMEM((STAGE_ROWS, D), jnp.float32),
        pltpu.VMEM((B,), jnp.float32),
        pltpu.VMEM((ROWS_PER_TILE,), jnp.float32),
        pltpu.VMEM_SHARED((N_NODES, D), jnp.float32),
        pltpu.VMEM_SHARED((N_NODES,), jnp.float32),
        pltpu.SemaphoreType.DMA,
        pltpu.SemaphoreType.DMA,
        pltpu.SemaphoreType.DMA,
        pltpu.SemaphoreType.DMA,
    ],
)
def _sc_aggregate(nbr_hbm, idx_hbm, z_feat_hbm, ones1d_hbm, z_cnt1d_hbm,
                  sum_out, cnt_out, idx_v, rows_v, stage_v, ones_v,
                  cnt_stage_v, acc_sh, cnt_sh, sem0, sem1, semc, semz):
    sid = lax.axis_index("s")
    wid = sid
    base_n = sid * ROWS_PER_TILE

    def zero_acc():
        zs = [pltpu.make_async_copy(stage_v, acc_sh.at[pl.ds(base_n + k * STAGE_ROWS, STAGE_ROWS)], semz)
              for k in range(ROWS_PER_TILE // STAGE_ROWS)]
        zc = pltpu.make_async_copy(cnt_stage_v,
                                   cnt_sh.at[pl.ds(base_n, ROWS_PER_TILE)], semz)
        for z in zs:
            z.start()
        zc.start()
        for z in zs:
            z.wait()
        zc.wait()

        @pl.when(sid == 15)
        def _zero_rem():
            pltpu.sync_copy(stage_v.at[pl.ds(0, REM_ROWS)],
                            acc_sh.at[pl.ds(REM_BASE, REM_ROWS)])
            pltpu.sync_copy(cnt_stage_v.at[pl.ds(0, REM_ROWS)],
                            cnt_sh.at[pl.ds(REM_BASE, REM_ROWS)])

    def copy_acc_out(out_hbm):
        for k in range(ROWS_PER_TILE // STAGE_ROWS):
            o = base_n + k * STAGE_ROWS
            pltpu.sync_copy(acc_sh.at[pl.ds(o, STAGE_ROWS)], stage_v)
            pltpu.sync_copy(stage_v, out_hbm.at[pl.ds(o, STAGE_ROWS)])

        @pl.when(sid == 15)
        def _copy_rem():
            pltpu.sync_copy(acc_sh.at[pl.ds(REM_BASE, REM_ROWS)],
                            stage_v.at[pl.ds(0, REM_ROWS)])
            pltpu.sync_copy(stage_v.at[pl.ds(0, REM_ROWS)],
                            out_hbm.at[pl.ds(REM_BASE, REM_ROWS)])

    def idx_copy(e, slot, sem):
        return pltpu.make_async_copy(
            idx_hbm.at[pl.ds(wid * E_PER_TILE + e * B, B)],
            idx_v.at[slot], sem)

    def row_copy(e, slot, sem):
        return pltpu.make_async_copy(nbr_hbm.at[wid, e], rows_v.at[slot], sem)

    idx_copy(0, 0, sem0).start()
    row_copy(0, 0, sem0).start()
    pltpu.sync_copy(z_feat_hbm, stage_v)
    pltpu.sync_copy(ones1d_hbm, ones_v)
    pltpu.sync_copy(z_cnt1d_hbm, cnt_stage_v)
    zero_acc()
    plsc.subcore_barrier()

    def body_feat(j2, carry):
        e0 = 2 * j2
        idx_copy(e0 + 1, 1, sem1).start()
        row_copy(e0 + 1, 1, sem1).start()
        idx_copy(e0, 0, sem0).wait()
        cnt0 = pltpu.async_copy(ones_v, cnt_sh.at[idx_v.at[0]], semc,
                                add=True)
        row_copy(e0, 0, sem0).wait()
        pltpu.sync_copy(rows_v.at[0], acc_sh.at[idx_v.at[0]], add=True)
        cnt0.wait()

        @pl.when(j2 + 1 < C // 2)
        def _pref():
            idx_copy(e0 + 2, 0, sem0).start()
            row_copy(e0 + 2, 0, sem0).start()

        idx_copy(e0 + 1, 1, sem1).wait()
        cnt1 = pltpu.async_copy(ones_v, cnt_sh.at[idx_v.at[1]], semc,
                                add=True)
        row_copy(e0 + 1, 1, sem1).wait()
        pltpu.sync_copy(rows_v.at[1], acc_sh.at[idx_v.at[1]], add=True)
        cnt1.wait()
        return carry

    lax.fori_loop(0, C // 2, body_feat, 0)
    plsc.subcore_barrier()
    copy_acc_out(sum_out)
    pltpu.sync_copy(cnt_sh.at[pl.ds(base_n, ROWS_PER_TILE)], cnt_stage_v)
    pltpu.sync_copy(cnt_stage_v, cnt_out.at[pl.ds(base_n, ROWS_PER_TILE)])

    @pl.when(sid == 15)
    def _copy_cnt_rem():
        pltpu.sync_copy(cnt_sh.at[pl.ds(REM_BASE, REM_ROWS)],
                        cnt_stage_v.at[pl.ds(0, REM_ROWS)])
        pltpu.sync_copy(cnt_stage_v.at[pl.ds(0, REM_ROWS)],
                        cnt_out.at[pl.ds(REM_BASE, REM_ROWS)])


_TC_BLOCK = 1000


def _tc_body(self_ref, s_ref, c_ref, w_ref, o_ref):
    deg = c_ref[...]
    x = (self_ref[...] + s_ref[...]) / (deg + 1.0)
    o_ref[...] = jnp.dot(x, w_ref[...], preferred_element_type=jnp.float32)


def kernel(self_feat, nbr_feat, relation_src_indices, W):
    idx = relation_src_indices.astype(jnp.int32)
    nbr = nbr_feat.reshape(NW, C, B, D)
    z_feat = jnp.zeros((STAGE_ROWS, D), jnp.float32)
    ones1d = jnp.ones((B,), jnp.float32)
    z_cnt1d = jnp.zeros((ROWS_PER_TILE,), jnp.float32)

    sums, cnts = _sc_aggregate(nbr, idx, z_feat, ones1d, z_cnt1d)
    cnts = cnts.reshape(N_NODES, 1)

    out = pl.pallas_call(
        _tc_body,
        grid=(N_NODES // _TC_BLOCK,),
        in_specs=[
            pl.BlockSpec((_TC_BLOCK, D), lambda i: (i, 0)),
            pl.BlockSpec((_TC_BLOCK, D), lambda i: (i, 0)),
            pl.BlockSpec((_TC_BLOCK, 1), lambda i: (i, 0)),
            pl.BlockSpec((D, D), lambda i: (0, 0)),
        ],
        out_specs=pl.BlockSpec((_TC_BLOCK, D), lambda i: (i, 0)),
        out_shape=jax.ShapeDtypeStruct((N_NODES, D), jnp.float32),
    )(self_feat, sums, cnts, W)
    return out
